# Initial kernel scaffold; baseline (speedup 1.0000x reference)
import jax
import jax.numpy as jnp
from jax.experimental import pallas as pl


def kernel(x, z, params):
    raise NotImplementedError("write your pallas kernel here")


# speedup vs baseline: 2.9436x; 2.9436x over previous
import functools

import jax
import jax.numpy as jnp
from jax.experimental import pallas as pl
from jax.experimental.pallas import tpu as pltpu

NEG = 0.01
NEG2 = 0.2
EPS = 1e-5
_BN = 1.0 / float(jnp.sqrt(jnp.float32(1.0 + EPS)))
HI = jax.lax.Precision.HIGHEST


def _leaky(v, s):
    return jnp.where(v >= 0, v, s * v)


def _dm(a, b):
    return jnp.dot(a, b, precision=HI, preferred_element_type=jnp.float32)


def _dbf(a, b):
    return jnp.dot(a.astype(jnp.bfloat16), b.astype(jnp.bfloat16),
                   preferred_element_type=jnp.float32)


def _dbft(a, b):
    return jax.lax.dot_general(
        a.astype(jnp.bfloat16), b.astype(jnp.bfloat16),
        (((1,), (1,)), ((), ())), preferred_element_type=jnp.float32)


def _head_kernel(x_ref, z_ref, h1a_ref, h1b_ref, h1bb_ref, h2t_ref, h2b_ref,
                 p1t_ref, p1b_ref, p2t_ref, p2b_ref, style_ref, pc_ref):
    x = x_ref[0]
    z = z_ref[0]
    zn = z / (jnp.sqrt(jnp.sum(z * z, axis=1, keepdims=True)) + 1e-8)
    s1 = _leaky(_dbf(x, h1a_ref[...]) + _dbf(zn, h1b_ref[...]) + h1bb_ref[...],
                NEG)
    style_ref[0] = _leaky(_dbf(s1, h2t_ref[...]) + h2b_ref[...], NEG)
    p1 = _leaky(_dbf(x, p1t_ref[...]) + p1b_ref[...], NEG)
    pc_ref[0] = _leaky(_dbf(p1, p2t_ref[...]) + p2b_ref[...], NEG)


def _edge_kernel(xin_ref, style_ref, w1t_ref, w1b_ref, w2t_ref, w2b_ref,
                 xct_ref, xnt_ref, xb_ref, ot_ref, ob_ref, at_ref, ab_ref,
                 out_ref, acc_ref, db_ref, nb_ref, w_ref, xx_ref, *, blk, k):
    xin = xin_ref[0]
    n = xin.shape[0]
    cdim0 = xin.shape[1]
    sqt = jnp.sum(xin * xin, axis=1, keepdims=True).T

    w1t = w1t_ref[...]; w1b = w1b_ref[...]
    w2t = w2t_ref[...]; w2b = w2b_ref[...]
    xct = xct_ref[...]; xnt = xnt_ref[...]; xb = xb_ref[...]
    ob = ob_ref[...]
    iota = jax.lax.broadcasted_iota(jnp.int32, (blk, n), 1)

    def body(i, carry):
        r0 = i * blk
        c = xin_ref[0, pl.ds(r0, blk), :]
        sqb = jnp.sum(c * c, axis=1, keepdims=True)
        db_ref[...] = sqb + sqt - 2.0 * _dbft(c, xin)

        def extract(j, carry2):
            db = db_ref[...]
            mval = jnp.min(db, axis=1, keepdims=True)
            idx = jnp.min(jnp.where(db == mval, iota, n), axis=1, keepdims=True)
            onehot = iota == idx

            @pl.when(j > 0)
            def _():
                nb_ref[pl.ds(j - 1, 1)] = _dm(onehot.astype(jnp.float32),
                                              xin)[None]

            db_ref[...] = jnp.where(onehot, 1e30, db)
            return carry2

        jax.lax.fori_loop(0, k + 1, extract, 0)
        xc = _dbf(c, xct)

        def conv1(j, mx):
            df = nb_ref[pl.ds(j, 1)][0] - c
            w = _leaky((_dbf(df, w1t) + w1b) * _BN, NEG)
            w = _leaky((_dbf(w, w2t) + w2b) * _BN, NEG)
            w_ref[pl.ds(j, 1)] = w[None]
            xx_ref[pl.ds(j, 1)] = _leaky((xc + _dbf(df, xnt) + xb) * _BN,
                                         NEG)[None]
            return jnp.maximum(mx, w)

        mx = jax.lax.fori_loop(
            0, k, conv1, jnp.full((blk, xc.shape[1]), -1e30, jnp.float32))

        def conv2(j, ssum):
            e = jnp.exp(w_ref[pl.ds(j, 1)][0] - mx)
            w_ref[pl.ds(j, 1)] = e[None]
            return ssum + e

        ssum = jax.lax.fori_loop(
            0, k, conv2, jnp.zeros((blk, xc.shape[1]), jnp.float32))

        def conv3(j, out):
            xxw = xx_ref[pl.ds(j, 1)][0] * (w_ref[pl.ds(j, 1)][0] / ssum)
            return out + _dbf(xxw, ot_ref[pl.ds(j, 1)][0])

        out = jax.lax.fori_loop(
            0, k, conv3, jnp.zeros((blk, xc.shape[1]), jnp.float32))
        acc_ref[pl.ds(r0, blk), :] = out + ob
        return carry

    jax.lax.fori_loop(0, n // blk, body, 0)

    x1 = _leaky(acc_ref[...], NEG2)
    m = jnp.mean(x1, axis=0, keepdims=True)
    v = jnp.mean((x1 - m) ** 2, axis=0, keepdims=True)
    xh = (x1 - m) / jnp.sqrt(v + EPS)
    s = _dbf(style_ref[0], at_ref[...]) + ab_ref[...]
    cdim = x1.shape[1]
    out_ref[0] = s[:, :cdim] * xh + s[:, cdim:]


def _tail_kernel(x2_ref, g1t_ref, g1b_ref, g2t_ref, g2b_ref, ata_ref, atb_ref,
                 apa_ref, apb_ref, aga_ref, agb_ref, aota_ref, aotb_ref,
                 m1a_ref, m1fg_ref, m1b_ref, gam_ref, m2t_ref, m2b_ref,
                 m3t_ref, m3b_ref, wh_ref, bh_ref, out_ref):
    x2 = x2_ref[0]
    n = x2.shape[0]
    gam = gam_ref[...]
    fgm = jnp.max(x2, axis=0, keepdims=True)
    fg1 = _leaky((_dbf(fgm, g1t_ref[...]) + g1b_ref[...]) * _BN, NEG)
    fg2 = _leaky((_dbf(fg1, g2t_ref[...]) + g2b_ref[...]) * _BN, NEG)
    theta = _dbf(x2, ata_ref[...]) + _dbf(fg2, atb_ref[...])
    phi = _dbf(x2, apa_ref[...]) + _dbf(fg2, apb_ref[...])
    gv = _dbf(x2, aga_ref[...]) + _dbf(fg2, agb_ref[...])
    logits = _dbft(theta, phi)
    e = jnp.exp(logits - jnp.max(logits, axis=1, keepdims=True))
    beta = e / jnp.sum(e, axis=1, keepdims=True)
    o_pre = _dbf(beta, gv)
    feat_a = gam * _dbf(o_pre, aota_ref[...]) + x2
    feat_b = (gam * _dbf(o_pre, aotb_ref[...])
              + jnp.broadcast_to(fg2, (n, fg2.shape[1])))
    m1in = _dbf(feat_a, m1a_ref[...]) + _dbf(feat_b, m1fg_ref[...]) + m1b_ref[...]
    h = jax.nn.silu(m1in)
    h = jax.nn.silu(_dbf(h, m2t_ref[...]) + m2b_ref[...])
    h = jax.nn.silu(_dbf(h, m3t_ref[...]) + m3b_ref[...])
    raw = _dbf(h, wh_ref[...]) + bh_ref[...]
    col = jax.lax.broadcasted_iota(jnp.int32, raw.shape, 1)
    sc = jnp.clip(jnp.exp(raw), 0.0, 0.02)
    nrm = jnp.sqrt(jnp.sum(raw * raw, axis=0, keepdims=True))
    rot = raw / jnp.maximum(nrm, 1e-12)
    sg = jax.nn.sigmoid(raw)
    xyz = jnp.tanh(raw * 0.1) * 0.6
    out_ref[0] = jnp.where(col < 3, sc,
                 jnp.where(col < 7, rot,
                 jnp.where(col < 11, sg, xyz)))


def _bb(shape):
    nz = len(shape)
    return pl.BlockSpec((1,) + shape, lambda b, _n=nz: (b,) + (0,) * _n)


def _bf(arr):
    nd = arr.ndim
    return pl.BlockSpec(arr.shape, lambda b, _n=nd: (0,) * _n)


_CP = pltpu.CompilerParams(vmem_limit_bytes=63 * 1024 * 1024)


def kernel(x, z, params):
    p = params
    B, N, _ = x.shape
    f32 = jnp.float32
    k = 16
    blk = 512 if N % 512 == 0 else N

    def t(w):
        return jnp.asarray(w, f32).T

    def rb(b):
        return jnp.asarray(b, f32).reshape(1, -1)

    h1a = t(p['h1_W'][:, :3]); h1b = t(p['h1_W'][:, 3:]); h1bb = rb(p['h1_b'])
    h2t = t(p['h2_W']); h2b = rb(p['h2_b'])
    p1t = t(p['p1_W']); p1b = rb(p['p1_b'])
    p2t = t(p['p2_W']); p2b = rb(p['p2_b'])

    def edge_w(pref, apref):
        return (t(p[pref + 'w1_W']), rb(p[pref + 'w1_b']),
                t(p[pref + 'w2_W']), rb(p[pref + 'w2_b']),
                t(p[pref + 'x_W'][:, :128]), t(p[pref + 'x_W'][:, 128:]),
                rb(p[pref + 'x_b']),
                jnp.transpose(p[pref + 'o_W'], (2, 1, 0)), rb(p[pref + 'o_b']),
                t(p[apref + '_W']), rb(p[apref + '_b']))

    g1t = t(p['g1_W']); g1b = rb(p['g1_b'])
    g2t = t(p['g2_W']); g2b = rb(p['g2_b'])
    ata = t(p['at_W'][:, :128]); atb = t(p['at_W'][:, 128:])
    apa = t(p['ap_W'][:, :128]); apb = t(p['ap_W'][:, 128:])
    aga = t(p['ag_W'][:, :128]); agb = t(p['ag_W'][:, 128:])
    aota = t(p['ao_W'][:128, :]); aotb = t(p['ao_W'][128:, :])
    m1a = t(p['m1_W'][:, :128]); m1fg = t(p['m1_W'][:, 128:]); m1b = rb(p['m1_b'])
    gam = jnp.asarray(p['a_gamma'], f32).reshape(1, 1)
    m2t = t(p['m2_W']); m2b = rb(p['m2_b'])
    m3t = t(p['m3_W']); m3b = rb(p['m3_b'])
    wh = t(jnp.concatenate([p['ds_W'], p['dr_W'], p['do_W'], p['dh_W'],
                            p['dx_W']], axis=0))
    bh = rb(jnp.concatenate([p['ds_b'], p['dr_b'], p['do_b'], p['dh_b'],
                             p['dx_b']], axis=0))

    head_in = (x, z, h1a, h1b, h1bb, h2t, h2b, p1t, p1b, p2t, p2b)
    style, pc = pl.pallas_call(
        _head_kernel,
        grid=(B,),
        in_specs=[_bb((N, 3)), _bb((N, 128))] + [_bf(a) for a in head_in[2:]],
        out_specs=[_bb((N, 128)), _bb((N, 128))],
        out_shape=[jax.ShapeDtypeStruct((B, N, 128), f32)] * 2,
        compiler_params=_CP,
    )(*head_in)

    def edge(xin, wts):
        ins = (xin, style) + wts
        return pl.pallas_call(
            functools.partial(_edge_kernel, blk=blk, k=k),
            grid=(B,),
            in_specs=[_bb((N, 128)), _bb((N, 128))] + [_bf(a) for a in wts],
            out_specs=_bb((N, 128)),
            out_shape=jax.ShapeDtypeStruct((B, N, 128), f32),
            scratch_shapes=[pltpu.VMEM((N, 128), f32),
                            pltpu.VMEM((blk, N), f32),
                            pltpu.VMEM((k, blk, 128), f32),
                            pltpu.VMEM((k, blk, 128), f32),
                            pltpu.VMEM((k, blk, 128), f32)],
            compiler_params=_CP,
        )(*ins)

    x1 = edge(pc, edge_w('e1_', 'a1'))
    x2 = edge(x1, edge_w('e2_', 'a2'))

    tail_w = (g1t, g1b, g2t, g2b, ata, atb, apa, apb, aga, agb, aota, aotb,
              m1a, m1fg, m1b, gam, m2t, m2b, m3t, m3b, wh, bh)
    out = pl.pallas_call(
        _tail_kernel,
        grid=(B,),
        in_specs=[_bb((N, 128))] + [_bf(a) for a in tail_w],
        out_specs=_bb((N, 14)),
        out_shape=jax.ShapeDtypeStruct((B, N, 14), f32),
        compiler_params=_CP,
    )(x2, *tail_w)
    return out


# speedup vs baseline: 4.5404x; 1.5424x over previous
import functools

import jax
import jax.numpy as jnp
from jax.experimental import pallas as pl
from jax.experimental.pallas import tpu as pltpu

NEG = 0.01
NEG2 = 0.2
EPS = 1e-5
_BN = (1.0 + EPS) ** -0.5
HI = jax.lax.Precision.HIGHEST


def _leaky(v, s):
    return jnp.where(v >= 0, v, s * v)


def _dbp(a, b):
    return jnp.dot(a, b, preferred_element_type=jnp.float32)


def _dbf(a, b):
    return jnp.dot(a.astype(jnp.bfloat16), b.astype(jnp.bfloat16),
                   preferred_element_type=jnp.float32)


def _dbft(a, b):
    return jax.lax.dot_general(
        a.astype(jnp.bfloat16), b.astype(jnp.bfloat16),
        (((1,), (1,)), ((), ())), preferred_element_type=jnp.float32)


def _head_kernel(x_ref, z_ref, h1a_ref, h1b_ref, h1bb_ref, h2t_ref, h2b_ref,
                 p1t_ref, p1b_ref, p2t_ref, p2b_ref, style_ref, pc_ref):
    x = x_ref[0]
    z = z_ref[0]
    zn = z / (jnp.sqrt(jnp.sum(z * z, axis=1, keepdims=True)) + 1e-8)
    s1 = _leaky(_dbf(x, h1a_ref[...]) + _dbf(zn, h1b_ref[...]) + h1bb_ref[...],
                NEG)
    style_ref[0] = _leaky(_dbf(s1, h2t_ref[...]) + h2b_ref[...], NEG)
    p1 = _leaky(_dbf(x, p1t_ref[...]) + p1b_ref[...], NEG)
    pc_ref[0] = _leaky(_dbf(p1, p2t_ref[...]) + p2b_ref[...], NEG)


def _edge_kernel(xin_ref, style_ref, w1t_ref, w1b_ref, w2t_ref, w2b_ref,
                 xct_ref, xnt_ref, xb_ref, ot_ref, ob_ref, at_ref, ab_ref,
                 out_ref, acc_ref, db_ref, nb_ref, w_ref, xx_ref, tbl_ref,
                 *, blk, k):
    xin = xin_ref[0]
    n = xin.shape[0]
    sqt = jnp.sum(xin * xin, axis=1, keepdims=True).T

    hi = xin.astype(jnp.bfloat16)
    r1 = xin - hi.astype(jnp.float32)
    mid = r1.astype(jnp.bfloat16)
    lo = (r1 - mid.astype(jnp.float32)).astype(jnp.bfloat16)
    tbl_ref[0] = hi
    tbl_ref[1] = mid
    tbl_ref[2] = lo

    w1t = w1t_ref[...]; w1b = w1b_ref[...]
    w2t = w2t_ref[...]; w2b = w2b_ref[...]
    xct = xct_ref[...]; xnt = xnt_ref[...]; xb = xb_ref[...]
    ob = ob_ref[...]
    iota = jax.lax.broadcasted_iota(jnp.int32, (blk, n), 1)

    def body(i, carry):
        r0 = i * blk
        c = xin_ref[0, pl.ds(r0, blk), :]
        sqb = jnp.sum(c * c, axis=1, keepdims=True)
        db_ref[...] = sqb + sqt - 2.0 * _dbft(c, xin)

        def extract(j, carry2):
            db = db_ref[...]
            mval = jnp.min(db, axis=1, keepdims=True)
            idx = jnp.min(jnp.where(db == mval, iota, n), axis=1, keepdims=True)
            onehot = iota == idx

            @pl.when(j > 0)
            def _():
                ohb = onehot.astype(jnp.bfloat16)
                g = (_dbp(ohb, tbl_ref[0]) + _dbp(ohb, tbl_ref[1])
                     + _dbp(ohb, tbl_ref[2]))
                nb_ref[pl.ds(j - 1, 1)] = g[None]

            db_ref[...] = jnp.where(onehot, 1e30, db)
            return carry2

        jax.lax.fori_loop(0, k + 1, extract, 0)
        xc = _dbf(c, xct)

        def conv1(j, mx):
            df = nb_ref[pl.ds(j, 1)][0] - c
            w = _leaky((_dbf(df, w1t) + w1b) * _BN, NEG)
            w = _leaky((_dbf(w, w2t) + w2b) * _BN, NEG)
            w_ref[pl.ds(j, 1)] = w[None]
            xx_ref[pl.ds(j, 1)] = _leaky((xc + _dbf(df, xnt) + xb) * _BN,
                                         NEG)[None]
            return jnp.maximum(mx, w)

        mx = jax.lax.fori_loop(
            0, k, conv1, jnp.full((blk, xc.shape[1]), -1e30, jnp.float32))

        def conv2(j, ssum):
            e = jnp.exp(w_ref[pl.ds(j, 1)][0] - mx)
            w_ref[pl.ds(j, 1)] = e[None]
            return ssum + e

        ssum = jax.lax.fori_loop(
            0, k, conv2, jnp.zeros((blk, xc.shape[1]), jnp.float32))

        def conv3(j, out):
            xxw = xx_ref[pl.ds(j, 1)][0] * (w_ref[pl.ds(j, 1)][0] / ssum)
            return out + _dbf(xxw, ot_ref[pl.ds(j, 1)][0])

        out = jax.lax.fori_loop(
            0, k, conv3, jnp.zeros((blk, xc.shape[1]), jnp.float32))
        acc_ref[pl.ds(r0, blk), :] = out + ob
        return carry

    jax.lax.fori_loop(0, n // blk, body, 0)

    x1 = _leaky(acc_ref[...], NEG2)
    m = jnp.mean(x1, axis=0, keepdims=True)
    v = jnp.mean((x1 - m) ** 2, axis=0, keepdims=True)
    xh = (x1 - m) / jnp.sqrt(v + EPS)
    s = _dbf(style_ref[0], at_ref[...]) + ab_ref[...]
    cdim = x1.shape[1]
    out_ref[0] = s[:, :cdim] * xh + s[:, cdim:]


def _tail_kernel(x2_ref, g1t_ref, g1b_ref, g2t_ref, g2b_ref, ata_ref, atb_ref,
                 apa_ref, apb_ref, aga_ref, agb_ref, aota_ref, aotb_ref,
                 m1a_ref, m1fg_ref, m1b_ref, gam_ref, m2t_ref, m2b_ref,
                 m3t_ref, m3b_ref, wh_ref, bh_ref, out_ref):
    x2 = x2_ref[0]
    n = x2.shape[0]
    gam = gam_ref[...]
    fgm = jnp.max(x2, axis=0, keepdims=True)
    fg1 = _leaky((_dbf(fgm, g1t_ref[...]) + g1b_ref[...]) * _BN, NEG)
    fg2 = _leaky((_dbf(fg1, g2t_ref[...]) + g2b_ref[...]) * _BN, NEG)
    theta = _dbf(x2, ata_ref[...]) + _dbf(fg2, atb_ref[...])
    phi = _dbf(x2, apa_ref[...]) + _dbf(fg2, apb_ref[...])
    gv = _dbf(x2, aga_ref[...]) + _dbf(fg2, agb_ref[...])
    logits = _dbft(theta, phi)
    e = jnp.exp(logits - jnp.max(logits, axis=1, keepdims=True))
    beta = e / jnp.sum(e, axis=1, keepdims=True)
    o_pre = _dbf(beta, gv)
    feat_a = gam * _dbf(o_pre, aota_ref[...]) + x2
    feat_b = (gam * _dbf(o_pre, aotb_ref[...])
              + jnp.broadcast_to(fg2, (n, fg2.shape[1])))
    m1in = _dbf(feat_a, m1a_ref[...]) + _dbf(feat_b, m1fg_ref[...]) + m1b_ref[...]
    h = jax.nn.silu(m1in)
    h = jax.nn.silu(_dbf(h, m2t_ref[...]) + m2b_ref[...])
    h = jax.nn.silu(_dbf(h, m3t_ref[...]) + m3b_ref[...])
    raw = _dbf(h, wh_ref[...]) + bh_ref[...]
    col = jax.lax.broadcasted_iota(jnp.int32, raw.shape, 1)
    sc = jnp.clip(jnp.exp(raw), 0.0, 0.02)
    nrm = jnp.sqrt(jnp.sum(raw * raw, axis=0, keepdims=True))
    rot = raw / jnp.maximum(nrm, 1e-12)
    sg = jax.nn.sigmoid(raw)
    xyz = jnp.tanh(raw * 0.1) * 0.6
    out_ref[0] = jnp.where(col < 3, sc,
                 jnp.where(col < 7, rot,
                 jnp.where(col < 11, sg, xyz)))


def _bb(shape):
    nz = len(shape)
    return pl.BlockSpec((1,) + shape, lambda b, _n=nz: (b,) + (0,) * _n)


def _bf(arr):
    nd = arr.ndim
    return pl.BlockSpec(arr.shape, lambda b, _n=nd: (0,) * _n)


_CP = pltpu.CompilerParams(vmem_limit_bytes=63 * 1024 * 1024)


def kernel(x, z, params):
    p = params
    B, N, _ = x.shape
    f32 = jnp.float32
    k = 16
    blk = 512 if N % 512 == 0 else N

    def t(w):
        return jnp.asarray(w, f32).T

    def rb(b):
        return jnp.asarray(b, f32).reshape(1, -1)

    h1a = t(p['h1_W'][:, :3]); h1b = t(p['h1_W'][:, 3:]); h1bb = rb(p['h1_b'])
    h2t = t(p['h2_W']); h2b = rb(p['h2_b'])
    p1t = t(p['p1_W']); p1b = rb(p['p1_b'])
    p2t = t(p['p2_W']); p2b = rb(p['p2_b'])

    def edge_w(pref, apref):
        return (t(p[pref + 'w1_W']), rb(p[pref + 'w1_b']),
                t(p[pref + 'w2_W']), rb(p[pref + 'w2_b']),
                t(p[pref + 'x_W'][:, :128]), t(p[pref + 'x_W'][:, 128:]),
                rb(p[pref + 'x_b']),
                jnp.transpose(p[pref + 'o_W'], (2, 1, 0)), rb(p[pref + 'o_b']),
                t(p[apref + '_W']), rb(p[apref + '_b']))

    g1t = t(p['g1_W']); g1b = rb(p['g1_b'])
    g2t = t(p['g2_W']); g2b = rb(p['g2_b'])
    ata = t(p['at_W'][:, :128]); atb = t(p['at_W'][:, 128:])
    apa = t(p['ap_W'][:, :128]); apb = t(p['ap_W'][:, 128:])
    aga = t(p['ag_W'][:, :128]); agb = t(p['ag_W'][:, 128:])
    aota = t(p['ao_W'][:128, :]); aotb = t(p['ao_W'][128:, :])
    m1a = t(p['m1_W'][:, :128]); m1fg = t(p['m1_W'][:, 128:]); m1b = rb(p['m1_b'])
    gam = jnp.asarray(p['a_gamma'], f32).reshape(1, 1)
    m2t = t(p['m2_W']); m2b = rb(p['m2_b'])
    m3t = t(p['m3_W']); m3b = rb(p['m3_b'])
    wh = t(jnp.concatenate([p['ds_W'], p['dr_W'], p['do_W'], p['dh_W'],
                            p['dx_W']], axis=0))
    bh = rb(jnp.concatenate([p['ds_b'], p['dr_b'], p['do_b'], p['dh_b'],
                             p['dx_b']], axis=0))

    head_in = (x, z, h1a, h1b, h1bb, h2t, h2b, p1t, p1b, p2t, p2b)
    style, pc = pl.pallas_call(
        _head_kernel,
        grid=(B,),
        in_specs=[_bb((N, 3)), _bb((N, 128))] + [_bf(a) for a in head_in[2:]],
        out_specs=[_bb((N, 128)), _bb((N, 128))],
        out_shape=[jax.ShapeDtypeStruct((B, N, 128), f32)] * 2,
        compiler_params=_CP,
    )(*head_in)

    def edge(xin, wts):
        ins = (xin, style) + wts
        return pl.pallas_call(
            functools.partial(_edge_kernel, blk=blk, k=k),
            grid=(B,),
            in_specs=[_bb((N, 128)), _bb((N, 128))] + [_bf(a) for a in wts],
            out_specs=_bb((N, 128)),
            out_shape=jax.ShapeDtypeStruct((B, N, 128), f32),
            scratch_shapes=[pltpu.VMEM((N, 128), f32),
                            pltpu.VMEM((blk, N), f32),
                            pltpu.VMEM((k, blk, 128), f32),
                            pltpu.VMEM((k, blk, 128), f32),
                            pltpu.VMEM((k, blk, 128), f32),
                            pltpu.VMEM((3, N, 128), jnp.bfloat16)],
            compiler_params=_CP,
        )(*ins)

    x1 = edge(pc, edge_w('e1_', 'a1'))
    x2 = edge(x1, edge_w('e2_', 'a2'))

    tail_w = (g1t, g1b, g2t, g2b, ata, atb, apa, apb, aga, agb, aota, aotb,
              m1a, m1fg, m1b, gam, m2t, m2b, m3t, m3b, wh, bh)
    out = pl.pallas_call(
        _tail_kernel,
        grid=(B,),
        in_specs=[_bb((N, 128))] + [_bf(a) for a in tail_w],
        out_specs=_bb((N, 14)),
        out_shape=jax.ShapeDtypeStruct((B, N, 14), f32),
        compiler_params=_CP,
    )(x2, *tail_w)
    return out


# speedup vs baseline: 4.5656x; 1.0056x over previous
import functools

import jax
import jax.numpy as jnp
from jax.experimental import pallas as pl
from jax.experimental.pallas import tpu as pltpu

NEG = 0.01
NEG2 = 0.2
EPS = 1e-5
_BN = (1.0 + EPS) ** -0.5
HI = jax.lax.Precision.HIGHEST


def _leaky(v, s):
    return jnp.where(v >= 0, v, s * v)


def _dbp(a, b):
    return jnp.dot(a, b, preferred_element_type=jnp.float32)


def _dbf(a, b):
    return jnp.dot(a.astype(jnp.bfloat16), b.astype(jnp.bfloat16),
                   preferred_element_type=jnp.float32)


def _dbft(a, b):
    return jax.lax.dot_general(
        a.astype(jnp.bfloat16), b.astype(jnp.bfloat16),
        (((1,), (1,)), ((), ())), preferred_element_type=jnp.float32)


def _head_kernel(x_ref, z_ref, h1a_ref, h1b_ref, h1bb_ref, h2t_ref, h2b_ref,
                 p1t_ref, p1b_ref, p2t_ref, p2b_ref, style_ref, pc_ref):
    x = x_ref[0]
    z = z_ref[0]
    zn = z / (jnp.sqrt(jnp.sum(z * z, axis=1, keepdims=True)) + 1e-8)
    s1 = _leaky(_dbf(x, h1a_ref[...]) + _dbf(zn, h1b_ref[...]) + h1bb_ref[...],
                NEG)
    style_ref[0] = _leaky(_dbf(s1, h2t_ref[...]) + h2b_ref[...], NEG)
    p1 = _leaky(_dbf(x, p1t_ref[...]) + p1b_ref[...], NEG)
    pc_ref[0] = _leaky(_dbf(p1, p2t_ref[...]) + p2b_ref[...], NEG)


def _edge_kernel(xin_ref, style_ref, w1t_ref, w1b_ref, w2t_ref, w2b_ref,
                 xct_ref, xnt_ref, xb_ref, ot_ref, ob_ref, at_ref, ab_ref,
                 out_ref, acc_ref, db_ref, nb_ref, w_ref, xx_ref, tbl_ref,
                 *, blk, k):
    xin = xin_ref[0]
    n = xin.shape[0]
    sqt = jnp.sum(xin * xin, axis=1, keepdims=True).T

    hi = xin.astype(jnp.bfloat16)
    r1 = xin - hi.astype(jnp.float32)
    mid = r1.astype(jnp.bfloat16)
    lo = (r1 - mid.astype(jnp.float32)).astype(jnp.bfloat16)
    tbl_ref[0] = hi
    tbl_ref[1] = mid
    tbl_ref[2] = lo

    w1t = w1t_ref[...]; w1b = w1b_ref[...]
    w2t = w2t_ref[...]; w2b = w2b_ref[...]
    xct = xct_ref[...]; xnt = xnt_ref[...]; xb = xb_ref[...]
    ob = ob_ref[...]
    iota = jax.lax.broadcasted_iota(jnp.int32, (blk, n), 1)

    def body(i, carry):
        r0 = i * blk
        c = xin_ref[0, pl.ds(r0, blk), :]
        sqb = jnp.sum(c * c, axis=1, keepdims=True)
        db_ref[...] = sqb + sqt - 2.0 * _dbft(c, xin)

        def extract(j, carry2):
            db = db_ref[...]
            idx = jnp.argmin(db, axis=1)[:, None]
            onehot = iota == idx

            @pl.when(j > 0)
            def _():
                ohb = onehot.astype(jnp.bfloat16)
                g = (_dbp(ohb, tbl_ref[0]) + _dbp(ohb, tbl_ref[1])
                     + _dbp(ohb, tbl_ref[2]))
                nb_ref[pl.ds(j - 1, 1)] = g[None]

            db_ref[...] = jnp.where(onehot, 1e30, db)
            return carry2

        jax.lax.fori_loop(0, k + 1, extract, 0, unroll=2)
        xc = _dbf(c, xct)

        def conv1(j, mx):
            df = nb_ref[pl.ds(j, 1)][0] - c
            w = _leaky((_dbf(df, w1t) + w1b) * _BN, NEG)
            w = _leaky((_dbf(w, w2t) + w2b) * _BN, NEG)
            w_ref[pl.ds(j, 1)] = w[None]
            xx_ref[pl.ds(j, 1)] = _leaky((xc + _dbf(df, xnt) + xb) * _BN,
                                         NEG)[None]
            return jnp.maximum(mx, w)

        mx = jax.lax.fori_loop(
            0, k, conv1, jnp.full((blk, xc.shape[1]), -1e30, jnp.float32))

        def conv2(j, ssum):
            e = jnp.exp(w_ref[pl.ds(j, 1)][0] - mx)
            w_ref[pl.ds(j, 1)] = e[None]
            return ssum + e

        ssum = jax.lax.fori_loop(
            0, k, conv2, jnp.zeros((blk, xc.shape[1]), jnp.float32))

        def conv3(j, out):
            xxw = xx_ref[pl.ds(j, 1)][0] * (w_ref[pl.ds(j, 1)][0] / ssum)
            return out + _dbf(xxw, ot_ref[pl.ds(j, 1)][0])

        out = jax.lax.fori_loop(
            0, k, conv3, jnp.zeros((blk, xc.shape[1]), jnp.float32))
        acc_ref[pl.ds(r0, blk), :] = out + ob
        return carry

    jax.lax.fori_loop(0, n // blk, body, 0)

    x1 = _leaky(acc_ref[...], NEG2)
    m = jnp.mean(x1, axis=0, keepdims=True)
    v = jnp.mean((x1 - m) ** 2, axis=0, keepdims=True)
    xh = (x1 - m) / jnp.sqrt(v + EPS)
    s = _dbf(style_ref[0], at_ref[...]) + ab_ref[...]
    cdim = x1.shape[1]
    out_ref[0] = s[:, :cdim] * xh + s[:, cdim:]


def _tail_kernel(x2_ref, g1t_ref, g1b_ref, g2t_ref, g2b_ref, ata_ref, atb_ref,
                 apa_ref, apb_ref, aga_ref, agb_ref, aota_ref, aotb_ref,
                 m1a_ref, m1fg_ref, m1b_ref, gam_ref, m2t_ref, m2b_ref,
                 m3t_ref, m3b_ref, wh_ref, bh_ref, out_ref):
    x2 = x2_ref[0]
    n = x2.shape[0]
    gam = gam_ref[...]
    fgm = jnp.max(x2, axis=0, keepdims=True)
    fg1 = _leaky((_dbf(fgm, g1t_ref[...]) + g1b_ref[...]) * _BN, NEG)
    fg2 = _leaky((_dbf(fg1, g2t_ref[...]) + g2b_ref[...]) * _BN, NEG)
    theta = _dbf(x2, ata_ref[...]) + _dbf(fg2, atb_ref[...])
    phi = _dbf(x2, apa_ref[...]) + _dbf(fg2, apb_ref[...])
    gv = _dbf(x2, aga_ref[...]) + _dbf(fg2, agb_ref[...])
    logits = _dbft(theta, phi)
    e = jnp.exp(logits - jnp.max(logits, axis=1, keepdims=True))
    beta = e / jnp.sum(e, axis=1, keepdims=True)
    o_pre = _dbf(beta, gv)
    feat_a = gam * _dbf(o_pre, aota_ref[...]) + x2
    feat_b = (gam * _dbf(o_pre, aotb_ref[...])
              + jnp.broadcast_to(fg2, (n, fg2.shape[1])))
    m1in = _dbf(feat_a, m1a_ref[...]) + _dbf(feat_b, m1fg_ref[...]) + m1b_ref[...]
    h = jax.nn.silu(m1in)
    h = jax.nn.silu(_dbf(h, m2t_ref[...]) + m2b_ref[...])
    h = jax.nn.silu(_dbf(h, m3t_ref[...]) + m3b_ref[...])
    raw = _dbf(h, wh_ref[...]) + bh_ref[...]
    col = jax.lax.broadcasted_iota(jnp.int32, raw.shape, 1)
    sc = jnp.clip(jnp.exp(raw), 0.0, 0.02)
    nrm = jnp.sqrt(jnp.sum(raw * raw, axis=0, keepdims=True))
    rot = raw / jnp.maximum(nrm, 1e-12)
    sg = jax.nn.sigmoid(raw)
    xyz = jnp.tanh(raw * 0.1) * 0.6
    out_ref[0] = jnp.where(col < 3, sc,
                 jnp.where(col < 7, rot,
                 jnp.where(col < 11, sg, xyz)))


def _bb(shape):
    nz = len(shape)
    return pl.BlockSpec((1,) + shape, lambda b, _n=nz: (b,) + (0,) * _n)


def _bf(arr):
    nd = arr.ndim
    return pl.BlockSpec(arr.shape, lambda b, _n=nd: (0,) * _n)


_CP = pltpu.CompilerParams(vmem_limit_bytes=63 * 1024 * 1024)


def kernel(x, z, params):
    p = params
    B, N, _ = x.shape
    f32 = jnp.float32
    k = 16
    blk = 512 if N % 512 == 0 else N

    def t(w):
        return jnp.asarray(w, f32).T

    def rb(b):
        return jnp.asarray(b, f32).reshape(1, -1)

    h1a = t(p['h1_W'][:, :3]); h1b = t(p['h1_W'][:, 3:]); h1bb = rb(p['h1_b'])
    h2t = t(p['h2_W']); h2b = rb(p['h2_b'])
    p1t = t(p['p1_W']); p1b = rb(p['p1_b'])
    p2t = t(p['p2_W']); p2b = rb(p['p2_b'])

    def edge_w(pref, apref):
        return (t(p[pref + 'w1_W']), rb(p[pref + 'w1_b']),
                t(p[pref + 'w2_W']), rb(p[pref + 'w2_b']),
                t(p[pref + 'x_W'][:, :128]), t(p[pref + 'x_W'][:, 128:]),
                rb(p[pref + 'x_b']),
                jnp.transpose(p[pref + 'o_W'], (2, 1, 0)), rb(p[pref + 'o_b']),
                t(p[apref + '_W']), rb(p[apref + '_b']))

    g1t = t(p['g1_W']); g1b = rb(p['g1_b'])
    g2t = t(p['g2_W']); g2b = rb(p['g2_b'])
    ata = t(p['at_W'][:, :128]); atb = t(p['at_W'][:, 128:])
    apa = t(p['ap_W'][:, :128]); apb = t(p['ap_W'][:, 128:])
    aga = t(p['ag_W'][:, :128]); agb = t(p['ag_W'][:, 128:])
    aota = t(p['ao_W'][:128, :]); aotb = t(p['ao_W'][128:, :])
    m1a = t(p['m1_W'][:, :128]); m1fg = t(p['m1_W'][:, 128:]); m1b = rb(p['m1_b'])
    gam = jnp.asarray(p['a_gamma'], f32).reshape(1, 1)
    m2t = t(p['m2_W']); m2b = rb(p['m2_b'])
    m3t = t(p['m3_W']); m3b = rb(p['m3_b'])
    wh = t(jnp.concatenate([p['ds_W'], p['dr_W'], p['do_W'], p['dh_W'],
                            p['dx_W']], axis=0))
    bh = rb(jnp.concatenate([p['ds_b'], p['dr_b'], p['do_b'], p['dh_b'],
                             p['dx_b']], axis=0))

    head_in = (x, z, h1a, h1b, h1bb, h2t, h2b, p1t, p1b, p2t, p2b)
    style, pc = pl.pallas_call(
        _head_kernel,
        grid=(B,),
        in_specs=[_bb((N, 3)), _bb((N, 128))] + [_bf(a) for a in head_in[2:]],
        out_specs=[_bb((N, 128)), _bb((N, 128))],
        out_shape=[jax.ShapeDtypeStruct((B, N, 128), f32)] * 2,
        compiler_params=_CP,
    )(*head_in)

    def edge(xin, wts):
        ins = (xin, style) + wts
        return pl.pallas_call(
            functools.partial(_edge_kernel, blk=blk, k=k),
            grid=(B,),
            in_specs=[_bb((N, 128)), _bb((N, 128))] + [_bf(a) for a in wts],
            out_specs=_bb((N, 128)),
            out_shape=jax.ShapeDtypeStruct((B, N, 128), f32),
            scratch_shapes=[pltpu.VMEM((N, 128), f32),
                            pltpu.VMEM((blk, N), f32),
                            pltpu.VMEM((k, blk, 128), f32),
                            pltpu.VMEM((k, blk, 128), f32),
                            pltpu.VMEM((k, blk, 128), f32),
                            pltpu.VMEM((3, N, 128), jnp.bfloat16)],
            compiler_params=_CP,
        )(*ins)

    x1 = edge(pc, edge_w('e1_', 'a1'))
    x2 = edge(x1, edge_w('e2_', 'a2'))

    tail_w = (g1t, g1b, g2t, g2b, ata, atb, apa, apb, aga, agb, aota, aotb,
              m1a, m1fg, m1b, gam, m2t, m2b, m3t, m3b, wh, bh)
    out = pl.pallas_call(
        _tail_kernel,
        grid=(B,),
        in_specs=[_bb((N, 128))] + [_bf(a) for a in tail_w],
        out_specs=_bb((N, 14)),
        out_shape=jax.ShapeDtypeStruct((B, N, 14), f32),
        compiler_params=_CP,
    )(x2, *tail_w)
    return out


# speedup vs baseline: 7.8779x; 1.7255x over previous
import functools

import jax
import jax.numpy as jnp
from jax.experimental import pallas as pl
from jax.experimental.pallas import tpu as pltpu

NEG = 0.01
NEG2 = 0.2
EPS = 1e-5
_BN = (1.0 + EPS) ** -0.5
HI = jax.lax.Precision.HIGHEST


def _leaky(v, s):
    return jnp.where(v >= 0, v, s * v)


def _dbp(a, b):
    return jnp.dot(a, b, preferred_element_type=jnp.float32)


def _dbf(a, b):
    return jnp.dot(a.astype(jnp.bfloat16), b.astype(jnp.bfloat16),
                   preferred_element_type=jnp.float32)


def _dbft(a, b):
    return jax.lax.dot_general(
        a.astype(jnp.bfloat16), b.astype(jnp.bfloat16),
        (((1,), (1,)), ((), ())), preferred_element_type=jnp.float32)


def _head_kernel(x_ref, z_ref, h1a_ref, h1b_ref, h1bb_ref, h2t_ref, h2b_ref,
                 p1t_ref, p1b_ref, p2t_ref, p2b_ref, style_ref, pc_ref):
    x = x_ref[0]
    z = z_ref[0]
    zn = z / (jnp.sqrt(jnp.sum(z * z, axis=1, keepdims=True)) + 1e-8)
    s1 = _leaky(_dbf(x, h1a_ref[...]) + _dbf(zn, h1b_ref[...]) + h1bb_ref[...],
                NEG)
    style_ref[0] = _leaky(_dbf(s1, h2t_ref[...]) + h2b_ref[...], NEG)
    p1 = _leaky(_dbf(x, p1t_ref[...]) + p1b_ref[...], NEG)
    pc_ref[0] = _leaky(_dbf(p1, p2t_ref[...]) + p2b_ref[...], NEG)


def _extract_kernel(xin_ref, idx_ref, db_ref, *, blk, k):
    xin = xin_ref[0]
    n = xin.shape[0]
    off = pl.program_id(0) * n
    sqt = jnp.sum(xin * xin, axis=1, keepdims=True).T
    iota = jax.lax.broadcasted_iota(jnp.int32, (blk, n), 1)
    lane = jax.lax.broadcasted_iota(jnp.int32, (blk, k), 1)

    def body(i, carry):
        r0 = i * blk
        c = xin_ref[0, pl.ds(r0, blk), :]
        sqb = jnp.sum(c * c, axis=1, keepdims=True)
        db_ref[...] = sqb + sqt - 2.0 * _dbft(c, xin)

        def extract(j, ib):
            db = db_ref[...]
            idx = jnp.argmin(db, axis=1)[:, None]
            onehot = iota == idx
            db_ref[...] = jnp.where(onehot, 1e30, db)
            return jnp.where(lane == j - 1,
                             jnp.broadcast_to(idx + off, (blk, k)), ib)

        ib = jax.lax.fori_loop(0, k + 1, extract,
                               jnp.zeros((blk, k), jnp.int32), unroll=2)
        idx_ref[0, pl.ds(r0, blk), :] = ib
        return carry

    jax.lax.fori_loop(0, n // blk, body, 0)


def _conv_kernel(nb_ref, xin_ref, w1t_ref, w1b_ref, w2t_ref, w2b_ref,
                 xct_ref, xnt_ref, xb_ref, ot_ref, ob_ref,
                 out_ref, w_ref, xx_ref, *, blk, k):
    c = xin_ref[0]
    w1t = w1t_ref[...]; w1b = w1b_ref[...]
    w2t = w2t_ref[...]; w2b = w2b_ref[...]
    xct = xct_ref[...]; xnt = xnt_ref[...]; xb = xb_ref[...]
    xc = _dbf(c, xct)

    def conv1(j, mx):
        df = nb_ref[0, pl.ds(j, 1)][0] - c
        w = _leaky((_dbf(df, w1t) + w1b) * _BN, NEG)
        w = _leaky((_dbf(w, w2t) + w2b) * _BN, NEG)
        w_ref[pl.ds(j, 1)] = w[None]
        xx_ref[pl.ds(j, 1)] = _leaky((xc + _dbf(df, xnt) + xb) * _BN,
                                     NEG)[None]
        return jnp.maximum(mx, w)

    mx = jax.lax.fori_loop(
        0, k, conv1, jnp.full((blk, xc.shape[1]), -1e30, jnp.float32))

    def conv2(j, ssum):
        e = jnp.exp(w_ref[pl.ds(j, 1)][0] - mx)
        w_ref[pl.ds(j, 1)] = e[None]
        return ssum + e

    ssum = jax.lax.fori_loop(
        0, k, conv2, jnp.zeros((blk, xc.shape[1]), jnp.float32))

    def conv3(j, out):
        xxw = xx_ref[pl.ds(j, 1)][0] * (w_ref[pl.ds(j, 1)][0] / ssum)
        return out + _dbf(xxw, ot_ref[pl.ds(j, 1)][0])

    out = jax.lax.fori_loop(
        0, k, conv3, jnp.zeros((blk, xc.shape[1]), jnp.float32))
    out_ref[0] = out + ob_ref[...]


def _adain_kernel(raw_ref, style_ref, at_ref, ab_ref, out_ref):
    x1 = _leaky(raw_ref[0], NEG2)
    m = jnp.mean(x1, axis=0, keepdims=True)
    v = jnp.mean((x1 - m) ** 2, axis=0, keepdims=True)
    xh = (x1 - m) / jnp.sqrt(v + EPS)
    s = _dbf(style_ref[0], at_ref[...]) + ab_ref[...]
    cdim = x1.shape[1]
    out_ref[0] = s[:, :cdim] * xh + s[:, cdim:]


def _sc_gather(table, idx):
    from jax.experimental.pallas import tpu_sc as plsc
    bt = idx.shape[0]
    d = table.shape[1]
    nc, ns = 2, 16
    nw = nc * ns
    b_per_w = bt // nw
    ch = min(512, b_per_w)
    nch = b_per_w // ch
    mesh = plsc.VectorSubcoreMesh(core_axis_name="c", subcore_axis_name="s")

    @functools.partial(
        pl.kernel, mesh=mesh,
        out_type=jax.ShapeDtypeStruct((bt, d), jnp.float32),
        scratch_types=[
            pltpu.VMEM((ch,), jnp.int32),
            pltpu.VMEM((ch, d), jnp.float32),
            pltpu.SemaphoreType.DMA,
        ],
    )
    def gk(table_hbm, idx_hbm, out_hbm, idx_v, rows_v, sem):
        wid = jax.lax.axis_index("s") * nc + jax.lax.axis_index("c")
        base = wid * b_per_w
        for ci in range(nch):
            o = base + ci * ch
            pltpu.sync_copy(idx_hbm.at[pl.ds(o, ch)], idx_v)
            pltpu.async_copy(table_hbm.at[idx_v], rows_v, sem).wait()
            pltpu.sync_copy(rows_v, out_hbm.at[pl.ds(o, ch)])

    return gk(table, idx)


def _tail_kernel(x2_ref, g1t_ref, g1b_ref, g2t_ref, g2b_ref, ata_ref, atb_ref,
                 apa_ref, apb_ref, aga_ref, agb_ref, aota_ref, aotb_ref,
                 m1a_ref, m1fg_ref, m1b_ref, gam_ref, m2t_ref, m2b_ref,
                 m3t_ref, m3b_ref, wh_ref, bh_ref, out_ref):
    x2 = x2_ref[0]
    n = x2.shape[0]
    gam = gam_ref[...]
    fgm = jnp.max(x2, axis=0, keepdims=True)
    fg1 = _leaky((_dbf(fgm, g1t_ref[...]) + g1b_ref[...]) * _BN, NEG)
    fg2 = _leaky((_dbf(fg1, g2t_ref[...]) + g2b_ref[...]) * _BN, NEG)
    theta = _dbf(x2, ata_ref[...]) + _dbf(fg2, atb_ref[...])
    phi = _dbf(x2, apa_ref[...]) + _dbf(fg2, apb_ref[...])
    gv = _dbf(x2, aga_ref[...]) + _dbf(fg2, agb_ref[...])
    logits = _dbft(theta, phi)
    e = jnp.exp(logits - jnp.max(logits, axis=1, keepdims=True))
    beta = e / jnp.sum(e, axis=1, keepdims=True)
    o_pre = _dbf(beta, gv)
    feat_a = gam * _dbf(o_pre, aota_ref[...]) + x2
    feat_b = (gam * _dbf(o_pre, aotb_ref[...])
              + jnp.broadcast_to(fg2, (n, fg2.shape[1])))
    m1in = _dbf(feat_a, m1a_ref[...]) + _dbf(feat_b, m1fg_ref[...]) + m1b_ref[...]
    h = jax.nn.silu(m1in)
    h = jax.nn.silu(_dbf(h, m2t_ref[...]) + m2b_ref[...])
    h = jax.nn.silu(_dbf(h, m3t_ref[...]) + m3b_ref[...])
    raw = _dbf(h, wh_ref[...]) + bh_ref[...]
    col = jax.lax.broadcasted_iota(jnp.int32, raw.shape, 1)
    sc = jnp.clip(jnp.exp(raw), 0.0, 0.02)
    nrm = jnp.sqrt(jnp.sum(raw * raw, axis=0, keepdims=True))
    rot = raw / jnp.maximum(nrm, 1e-12)
    sg = jax.nn.sigmoid(raw)
    xyz = jnp.tanh(raw * 0.1) * 0.6
    out_ref[0] = jnp.where(col < 3, sc,
                 jnp.where(col < 7, rot,
                 jnp.where(col < 11, sg, xyz)))


def _bb(shape):
    nz = len(shape)
    return pl.BlockSpec((1,) + shape, lambda b, _n=nz: (b,) + (0,) * _n)


def _bf(arr):
    nd = arr.ndim
    return pl.BlockSpec(arr.shape, lambda b, _n=nd: (0,) * _n)


_CP = pltpu.CompilerParams(vmem_limit_bytes=63 * 1024 * 1024)


def kernel(x, z, params):
    p = params
    B, N, _ = x.shape
    f32 = jnp.float32
    k = 16
    blk = 512 if N % 512 == 0 else N

    def t(w):
        return jnp.asarray(w, f32).T

    def rb(b):
        return jnp.asarray(b, f32).reshape(1, -1)

    h1a = t(p['h1_W'][:, :3]); h1b = t(p['h1_W'][:, 3:]); h1bb = rb(p['h1_b'])
    h2t = t(p['h2_W']); h2b = rb(p['h2_b'])
    p1t = t(p['p1_W']); p1b = rb(p['p1_b'])
    p2t = t(p['p2_W']); p2b = rb(p['p2_b'])

    def edge_w(pref, apref):
        return (t(p[pref + 'w1_W']), rb(p[pref + 'w1_b']),
                t(p[pref + 'w2_W']), rb(p[pref + 'w2_b']),
                t(p[pref + 'x_W'][:, :128]), t(p[pref + 'x_W'][:, 128:]),
                rb(p[pref + 'x_b']),
                jnp.transpose(p[pref + 'o_W'], (2, 1, 0)), rb(p[pref + 'o_b']),
                t(p[apref + '_W']), rb(p[apref + '_b']))

    g1t = t(p['g1_W']); g1b = rb(p['g1_b'])
    g2t = t(p['g2_W']); g2b = rb(p['g2_b'])
    ata = t(p['at_W'][:, :128]); atb = t(p['at_W'][:, 128:])
    apa = t(p['ap_W'][:, :128]); apb = t(p['ap_W'][:, 128:])
    aga = t(p['ag_W'][:, :128]); agb = t(p['ag_W'][:, 128:])
    aota = t(p['ao_W'][:128, :]); aotb = t(p['ao_W'][128:, :])
    m1a = t(p['m1_W'][:, :128]); m1fg = t(p['m1_W'][:, 128:]); m1b = rb(p['m1_b'])
    gam = jnp.asarray(p['a_gamma'], f32).reshape(1, 1)
    m2t = t(p['m2_W']); m2b = rb(p['m2_b'])
    m3t = t(p['m3_W']); m3b = rb(p['m3_b'])
    wh = t(jnp.concatenate([p['ds_W'], p['dr_W'], p['do_W'], p['dh_W'],
                            p['dx_W']], axis=0))
    bh = rb(jnp.concatenate([p['ds_b'], p['dr_b'], p['do_b'], p['dh_b'],
                             p['dx_b']], axis=0))

    head_in = (x, z, h1a, h1b, h1bb, h2t, h2b, p1t, p1b, p2t, p2b)
    style, pc = pl.pallas_call(
        _head_kernel,
        grid=(B,),
        in_specs=[_bb((N, 3)), _bb((N, 128))] + [_bf(a) for a in head_in[2:]],
        out_specs=[_bb((N, 128)), _bb((N, 128))],
        out_shape=[jax.ShapeDtypeStruct((B, N, 128), f32)] * 2,
        compiler_params=_CP,
    )(*head_in)

    def edge(xin, wts):
        idx = pl.pallas_call(
            functools.partial(_extract_kernel, blk=blk, k=k),
            grid=(B,),
            in_specs=[_bb((N, 128))],
            out_specs=_bb((N, k)),
            out_shape=jax.ShapeDtypeStruct((B, N, k), jnp.int32),
            scratch_shapes=[pltpu.VMEM((blk, N), f32)],
            compiler_params=_CP,
        )(xin)
        idx_flat = jnp.transpose(idx, (0, 2, 1)).reshape(-1)
        nb = _sc_gather(xin.reshape(B * N, 128), idx_flat)
        nb4 = nb.reshape(B, k, N, 128)
        nblk = N // blk
        raw = pl.pallas_call(
            functools.partial(_conv_kernel, blk=blk, k=k),
            grid=(B, nblk),
            in_specs=[pl.BlockSpec((1, k, blk, 128),
                                   lambda b, i: (b, 0, i, 0)),
                      pl.BlockSpec((1, blk, 128), lambda b, i: (b, i, 0))]
                     + [pl.BlockSpec(a.shape,
                                     lambda b, i, _n=a.ndim: (0,) * _n)
                        for a in wts[:9]],
            out_specs=pl.BlockSpec((1, blk, 128), lambda b, i: (b, i, 0)),
            out_shape=jax.ShapeDtypeStruct((B, N, 128), f32),
            scratch_shapes=[pltpu.VMEM((k, blk, 128), f32),
                            pltpu.VMEM((k, blk, 128), f32)],
            compiler_params=_CP,
        )(nb4, xin, *wts[:9])
        return pl.pallas_call(
            _adain_kernel,
            grid=(B,),
            in_specs=[_bb((N, 128)), _bb((N, 128)), _bf(wts[9]), _bf(wts[10])],
            out_specs=_bb((N, 128)),
            out_shape=jax.ShapeDtypeStruct((B, N, 128), f32),
            compiler_params=_CP,
        )(raw, style, wts[9], wts[10])

    x1 = edge(pc, edge_w('e1_', 'a1'))
    x2 = edge(x1, edge_w('e2_', 'a2'))

    tail_w = (g1t, g1b, g2t, g2b, ata, atb, apa, apb, aga, agb, aota, aotb,
              m1a, m1fg, m1b, gam, m2t, m2b, m3t, m3b, wh, bh)
    out = pl.pallas_call(
        _tail_kernel,
        grid=(B,),
        in_specs=[_bb((N, 128))] + [_bf(a) for a in tail_w],
        out_specs=_bb((N, 14)),
        out_shape=jax.ShapeDtypeStruct((B, N, 14), f32),
        compiler_params=_CP,
    )(x2, *tail_w)
    return out


# speedup vs baseline: 9.3204x; 1.1831x over previous
import functools

import jax
import jax.numpy as jnp
from jax.experimental import pallas as pl
from jax.experimental.pallas import tpu as pltpu

NEG = 0.01
NEG2 = 0.2
EPS = 1e-5
_BN = (1.0 + EPS) ** -0.5
HI = jax.lax.Precision.HIGHEST


def _leaky(v, s):
    return jnp.where(v >= 0, v, s * v)


def _dbp(a, b):
    return jnp.dot(a, b, preferred_element_type=jnp.float32)


def _dbf(a, b):
    return jnp.dot(a.astype(jnp.bfloat16), b.astype(jnp.bfloat16),
                   preferred_element_type=jnp.float32)


def _dbft(a, b):
    return jax.lax.dot_general(
        a.astype(jnp.bfloat16), b.astype(jnp.bfloat16),
        (((1,), (1,)), ((), ())), preferred_element_type=jnp.float32)


def _head_kernel(x_ref, z_ref, h1a_ref, h1b_ref, h1bb_ref, h2t_ref, h2b_ref,
                 p1t_ref, p1b_ref, p2t_ref, p2b_ref, style_ref, pc_ref):
    x = x_ref[0]
    z = z_ref[0]
    zn = z / (jnp.sqrt(jnp.sum(z * z, axis=1, keepdims=True)) + 1e-8)
    s1 = _leaky(_dbf(x, h1a_ref[...]) + _dbf(zn, h1b_ref[...]) + h1bb_ref[...],
                NEG)
    style_ref[0] = _leaky(_dbf(s1, h2t_ref[...]) + h2b_ref[...], NEG)
    p1 = _leaky(_dbf(x, p1t_ref[...]) + p1b_ref[...], NEG)
    pc_ref[0] = _leaky(_dbf(p1, p2t_ref[...]) + p2b_ref[...], NEG)


def _extract_kernel(xin_ref, idx_ref, db_ref, *, blk, k):
    xin = xin_ref[0]
    n = xin.shape[0]
    off = pl.program_id(0) * n
    sqt = jnp.sum(xin * xin, axis=1, keepdims=True).T
    iota = jax.lax.broadcasted_iota(jnp.int32, (blk, n), 1)
    lane = jax.lax.broadcasted_iota(jnp.int32, (blk, k), 1)

    def body(i, carry):
        r0 = i * blk
        c = xin_ref[0, pl.ds(r0, blk), :]
        sqb = jnp.sum(c * c, axis=1, keepdims=True)
        db_ref[...] = sqb + sqt - 2.0 * _dbft(c, xin)

        def extract(j, ib):
            db = db_ref[...]
            idx = jnp.argmin(db, axis=1)[:, None]
            onehot = iota == idx
            db_ref[...] = jnp.where(onehot, 1e30, db)
            return jnp.where(lane == j - 1,
                             jnp.broadcast_to(idx + off, (blk, k)), ib)

        ib = jax.lax.fori_loop(0, k + 1, extract,
                               jnp.zeros((blk, k), jnp.int32), unroll=2)
        idx_ref[0, pl.ds(r0, blk), :] = ib
        return carry

    jax.lax.fori_loop(0, n // blk, body, 0)


def _conv_kernel(nb_ref, xin_ref, w1t_ref, w1b_ref, w2t_ref, w2b_ref,
                 xct_ref, xnt_ref, xb_ref, otf_ref, ob_ref,
                 out_ref, *, blk, k):
    c = xin_ref[0]
    cd = c.shape[1]
    nb = nb_ref[0].reshape(k * blk, cd)
    cb = jnp.broadcast_to(c[None], (k, blk, cd)).reshape(k * blk, cd)
    df = nb - cb
    w = _leaky((_dbf(df, w1t_ref[...]) + w1b_ref[...]) * _BN, NEG)
    w = _leaky((_dbf(w, w2t_ref[...]) + w2b_ref[...]) * _BN, NEG)
    xc = jnp.broadcast_to(_dbf(c, xct_ref[...])[None],
                          (k, blk, cd)).reshape(k * blk, cd)
    xx = _leaky((xc + _dbf(df, xnt_ref[...]) + xb_ref[...]) * _BN, NEG)
    w3 = w.reshape(k, blk, cd)
    e = jnp.exp(w3 - jnp.max(w3, axis=0, keepdims=True))
    sm = e / jnp.sum(e, axis=0, keepdims=True)
    xxw = xx.reshape(k, blk, cd) * sm
    flat = jnp.swapaxes(xxw, 0, 1).reshape(blk, k * cd)
    out_ref[0] = _dbf(flat, otf_ref[...]) + ob_ref[...]


def _adain_kernel(raw_ref, style_ref, at_ref, ab_ref, out_ref):
    x1 = _leaky(raw_ref[0], NEG2)
    m = jnp.mean(x1, axis=0, keepdims=True)
    v = jnp.mean((x1 - m) ** 2, axis=0, keepdims=True)
    xh = (x1 - m) / jnp.sqrt(v + EPS)
    s = _dbf(style_ref[0], at_ref[...]) + ab_ref[...]
    cdim = x1.shape[1]
    out_ref[0] = s[:, :cdim] * xh + s[:, cdim:]


def _sc_gather(table, idx):
    from jax.experimental.pallas import tpu_sc as plsc
    bt = idx.shape[0]
    d = table.shape[1]
    nc, ns = 2, 16
    nw = nc * ns
    b_per_w = bt // nw
    ch = min(512, b_per_w)
    nch = b_per_w // ch
    mesh = plsc.VectorSubcoreMesh(core_axis_name="c", subcore_axis_name="s")

    @functools.partial(
        pl.kernel, mesh=mesh,
        out_type=jax.ShapeDtypeStruct((bt, d), jnp.float32),
        scratch_types=[
            pltpu.VMEM((ch,), jnp.int32),
            pltpu.VMEM((ch, d), jnp.float32),
            pltpu.SemaphoreType.DMA,
        ],
    )
    def gk(table_hbm, idx_hbm, out_hbm, idx_v, rows_v, sem):
        wid = jax.lax.axis_index("s") * nc + jax.lax.axis_index("c")
        base = wid * b_per_w
        for ci in range(nch):
            o = base + ci * ch
            pltpu.sync_copy(idx_hbm.at[pl.ds(o, ch)], idx_v)
            pltpu.async_copy(table_hbm.at[idx_v], rows_v, sem).wait()
            pltpu.sync_copy(rows_v, out_hbm.at[pl.ds(o, ch)])

    return gk(table, idx)


def _tail_kernel(x2_ref, g1t_ref, g1b_ref, g2t_ref, g2b_ref, ata_ref, atb_ref,
                 apa_ref, apb_ref, aga_ref, agb_ref, aota_ref, aotb_ref,
                 m1a_ref, m1fg_ref, m1b_ref, gam_ref, m2t_ref, m2b_ref,
                 m3t_ref, m3b_ref, wh_ref, bh_ref, out_ref):
    x2 = x2_ref[0]
    n = x2.shape[0]
    gam = gam_ref[...]
    fgm = jnp.max(x2, axis=0, keepdims=True)
    fg1 = _leaky((_dbf(fgm, g1t_ref[...]) + g1b_ref[...]) * _BN, NEG)
    fg2 = _leaky((_dbf(fg1, g2t_ref[...]) + g2b_ref[...]) * _BN, NEG)
    theta = _dbf(x2, ata_ref[...]) + _dbf(fg2, atb_ref[...])
    phi = _dbf(x2, apa_ref[...]) + _dbf(fg2, apb_ref[...])
    gv = _dbf(x2, aga_ref[...]) + _dbf(fg2, agb_ref[...])
    logits = _dbft(theta, phi)
    e = jnp.exp(logits - jnp.max(logits, axis=1, keepdims=True))
    beta = e / jnp.sum(e, axis=1, keepdims=True)
    o_pre = _dbf(beta, gv)
    feat_a = gam * _dbf(o_pre, aota_ref[...]) + x2
    feat_b = (gam * _dbf(o_pre, aotb_ref[...])
              + jnp.broadcast_to(fg2, (n, fg2.shape[1])))
    m1in = _dbf(feat_a, m1a_ref[...]) + _dbf(feat_b, m1fg_ref[...]) + m1b_ref[...]
    h = jax.nn.silu(m1in)
    h = jax.nn.silu(_dbf(h, m2t_ref[...]) + m2b_ref[...])
    h = jax.nn.silu(_dbf(h, m3t_ref[...]) + m3b_ref[...])
    raw = _dbf(h, wh_ref[...]) + bh_ref[...]
    col = jax.lax.broadcasted_iota(jnp.int32, raw.shape, 1)
    sc = jnp.clip(jnp.exp(raw), 0.0, 0.02)
    nrm = jnp.sqrt(jnp.sum(raw * raw, axis=0, keepdims=True))
    rot = raw / jnp.maximum(nrm, 1e-12)
    sg = jax.nn.sigmoid(raw)
    xyz = jnp.tanh(raw * 0.1) * 0.6
    out_ref[0] = jnp.where(col < 3, sc,
                 jnp.where(col < 7, rot,
                 jnp.where(col < 11, sg, xyz)))


def _bb(shape):
    nz = len(shape)
    return pl.BlockSpec((1,) + shape, lambda b, _n=nz: (b,) + (0,) * _n)


def _bf(arr):
    nd = arr.ndim
    return pl.BlockSpec(arr.shape, lambda b, _n=nd: (0,) * _n)


_CP = pltpu.CompilerParams(vmem_limit_bytes=63 * 1024 * 1024)


def kernel(x, z, params):
    p = params
    B, N, _ = x.shape
    f32 = jnp.float32
    k = 16
    blk = 512 if N % 512 == 0 else N

    def t(w):
        return jnp.asarray(w, f32).T

    def rb(b):
        return jnp.asarray(b, f32).reshape(1, -1)

    h1a = t(p['h1_W'][:, :3]); h1b = t(p['h1_W'][:, 3:]); h1bb = rb(p['h1_b'])
    h2t = t(p['h2_W']); h2b = rb(p['h2_b'])
    p1t = t(p['p1_W']); p1b = rb(p['p1_b'])
    p2t = t(p['p2_W']); p2b = rb(p['p2_b'])

    def edge_w(pref, apref):
        return (t(p[pref + 'w1_W']), rb(p[pref + 'w1_b']),
                t(p[pref + 'w2_W']), rb(p[pref + 'w2_b']),
                t(p[pref + 'x_W'][:, :128]), t(p[pref + 'x_W'][:, 128:]),
                rb(p[pref + 'x_b']),
                jnp.transpose(p[pref + 'o_W'], (2, 1, 0)).reshape(-1, 128),
                rb(p[pref + 'o_b']),
                t(p[apref + '_W']), rb(p[apref + '_b']))

    g1t = t(p['g1_W']); g1b = rb(p['g1_b'])
    g2t = t(p['g2_W']); g2b = rb(p['g2_b'])
    ata = t(p['at_W'][:, :128]); atb = t(p['at_W'][:, 128:])
    apa = t(p['ap_W'][:, :128]); apb = t(p['ap_W'][:, 128:])
    aga = t(p['ag_W'][:, :128]); agb = t(p['ag_W'][:, 128:])
    aota = t(p['ao_W'][:128, :]); aotb = t(p['ao_W'][128:, :])
    m1a = t(p['m1_W'][:, :128]); m1fg = t(p['m1_W'][:, 128:]); m1b = rb(p['m1_b'])
    gam = jnp.asarray(p['a_gamma'], f32).reshape(1, 1)
    m2t = t(p['m2_W']); m2b = rb(p['m2_b'])
    m3t = t(p['m3_W']); m3b = rb(p['m3_b'])
    wh = t(jnp.concatenate([p['ds_W'], p['dr_W'], p['do_W'], p['dh_W'],
                            p['dx_W']], axis=0))
    bh = rb(jnp.concatenate([p['ds_b'], p['dr_b'], p['do_b'], p['dh_b'],
                             p['dx_b']], axis=0))

    head_in = (x, z, h1a, h1b, h1bb, h2t, h2b, p1t, p1b, p2t, p2b)
    style, pc = pl.pallas_call(
        _head_kernel,
        grid=(B,),
        in_specs=[_bb((N, 3)), _bb((N, 128))] + [_bf(a) for a in head_in[2:]],
        out_specs=[_bb((N, 128)), _bb((N, 128))],
        out_shape=[jax.ShapeDtypeStruct((B, N, 128), f32)] * 2,
        compiler_params=_CP,
    )(*head_in)

    def edge(xin, wts):
        idx = pl.pallas_call(
            functools.partial(_extract_kernel, blk=blk, k=k),
            grid=(B,),
            in_specs=[_bb((N, 128))],
            out_specs=_bb((N, k)),
            out_shape=jax.ShapeDtypeStruct((B, N, k), jnp.int32),
            scratch_shapes=[pltpu.VMEM((blk, N), f32)],
            compiler_params=_CP,
        )(xin)
        idx_flat = jnp.transpose(idx, (0, 2, 1)).reshape(-1)
        nb = _sc_gather(xin.reshape(B * N, 128), idx_flat)
        nb4 = nb.reshape(B, k, N, 128)
        nblk = N // blk
        raw = pl.pallas_call(
            functools.partial(_conv_kernel, blk=blk, k=k),
            grid=(B, nblk),
            in_specs=[pl.BlockSpec((1, k, blk, 128),
                                   lambda b, i: (b, 0, i, 0)),
                      pl.BlockSpec((1, blk, 128), lambda b, i: (b, i, 0))]
                     + [pl.BlockSpec(a.shape,
                                     lambda b, i, _n=a.ndim: (0,) * _n)
                        for a in wts[:9]],
            out_specs=pl.BlockSpec((1, blk, 128), lambda b, i: (b, i, 0)),
            out_shape=jax.ShapeDtypeStruct((B, N, 128), f32),
            compiler_params=_CP,
        )(nb4, xin, *wts[:9])
        return pl.pallas_call(
            _adain_kernel,
            grid=(B,),
            in_specs=[_bb((N, 128)), _bb((N, 128)), _bf(wts[9]), _bf(wts[10])],
            out_specs=_bb((N, 128)),
            out_shape=jax.ShapeDtypeStruct((B, N, 128), f32),
            compiler_params=_CP,
        )(raw, style, wts[9], wts[10])

    x1 = edge(pc, edge_w('e1_', 'a1'))
    x2 = edge(x1, edge_w('e2_', 'a2'))

    tail_w = (g1t, g1b, g2t, g2b, ata, atb, apa, apb, aga, agb, aota, aotb,
              m1a, m1fg, m1b, gam, m2t, m2b, m3t, m3b, wh, bh)
    out = pl.pallas_call(
        _tail_kernel,
        grid=(B,),
        in_specs=[_bb((N, 128))] + [_bf(a) for a in tail_w],
        out_specs=_bb((N, 14)),
        out_shape=jax.ShapeDtypeStruct((B, N, 14), f32),
        compiler_params=_CP,
    )(x2, *tail_w)
    return out


# speedup vs baseline: 9.8138x; 1.0529x over previous
import functools

import jax
import jax.numpy as jnp
from jax.experimental import pallas as pl
from jax.experimental.pallas import tpu as pltpu

NEG = 0.01
NEG2 = 0.2
EPS = 1e-5
_BN = (1.0 + EPS) ** -0.5
HI = jax.lax.Precision.HIGHEST


def _leaky(v, s):
    return jnp.where(v >= 0, v, s * v)


def _dbp(a, b):
    return jnp.dot(a, b, preferred_element_type=jnp.float32)


def _dbf(a, b):
    return jnp.dot(a.astype(jnp.bfloat16), b.astype(jnp.bfloat16),
                   preferred_element_type=jnp.float32)


def _dbft(a, b):
    return jax.lax.dot_general(
        a.astype(jnp.bfloat16), b.astype(jnp.bfloat16),
        (((1,), (1,)), ((), ())), preferred_element_type=jnp.float32)


def _head_kernel(x_ref, z_ref, h1a_ref, h1b_ref, h1bb_ref, h2t_ref, h2b_ref,
                 p1t_ref, p1b_ref, p2t_ref, p2b_ref, style_ref, pc_ref):
    x = x_ref[0]
    z = z_ref[0]
    zn = z / (jnp.sqrt(jnp.sum(z * z, axis=1, keepdims=True)) + 1e-8)
    s1 = _leaky(_dbf(x, h1a_ref[...]) + _dbf(zn, h1b_ref[...]) + h1bb_ref[...],
                NEG)
    style_ref[0] = _leaky(_dbf(s1, h2t_ref[...]) + h2b_ref[...], NEG)
    p1 = _leaky(_dbf(x, p1t_ref[...]) + p1b_ref[...], NEG)
    pc_ref[0] = _leaky(_dbf(p1, p2t_ref[...]) + p2b_ref[...], NEG)


def _extract_kernel(xin_ref, idx_ref, db_ref, *, blk, k):
    xin = xin_ref[0]
    n = xin.shape[0]
    off = pl.program_id(0) * n
    sqt = jnp.sum(xin * xin, axis=1, keepdims=True).T
    iota = jax.lax.broadcasted_iota(jnp.int32, (blk, n), 1)
    lane = jax.lax.broadcasted_iota(jnp.int32, (blk, k), 1)

    def body(i, carry):
        r0 = i * blk
        c = xin_ref[0, pl.ds(r0, blk), :]
        sqb = jnp.sum(c * c, axis=1, keepdims=True)
        db_ref[...] = sqb + sqt - 2.0 * _dbft(c, xin)

        def extract(j, ib):
            db = db_ref[...]
            idx = jnp.argmin(db, axis=1)[:, None]
            onehot = iota == idx
            db_ref[...] = jnp.where(onehot, 1e30, db)
            return jnp.where(lane == j - 1,
                             jnp.broadcast_to(idx + off, (blk, k)), ib)

        ib = jax.lax.fori_loop(0, k + 1, extract,
                               jnp.zeros((blk, k), jnp.int32), unroll=2)
        idx_ref[0, pl.ds(r0, blk), :] = ib
        return carry

    jax.lax.fori_loop(0, n // blk, body, 0)


def _conv_kernel(nb_ref, xin_ref, w1t_ref, w1b_ref, w2t_ref, w2b_ref,
                 xct_ref, xnt_ref, xb_ref, otf_ref, ob_ref,
                 out_ref, *, blk, k):
    c = xin_ref[0]
    cd = c.shape[1]
    nb = nb_ref[0].reshape(k * blk, cd)
    cb = jnp.broadcast_to(c[None], (k, blk, cd)).reshape(k * blk, cd)
    df = nb - cb
    w = _leaky((_dbf(df, w1t_ref[...]) + w1b_ref[...]) * _BN, NEG)
    w = _leaky((_dbf(w, w2t_ref[...]) + w2b_ref[...]) * _BN, NEG)
    xc = jnp.broadcast_to(_dbf(c, xct_ref[...])[None],
                          (k, blk, cd)).reshape(k * blk, cd)
    xx = _leaky((xc + _dbf(df, xnt_ref[...]) + xb_ref[...]) * _BN, NEG)
    w3 = w.reshape(k, blk, cd)
    e = jnp.exp(w3 - jnp.max(w3, axis=0, keepdims=True))
    sm = e / jnp.sum(e, axis=0, keepdims=True)
    xxw = xx.reshape(k, blk, cd) * sm
    flat = jnp.swapaxes(xxw, 0, 1).reshape(blk, k * cd)
    out_ref[0] = _dbf(flat, otf_ref[...]) + ob_ref[...]


def _adain_kernel(raw_ref, style_ref, at_ref, ab_ref, out_ref):
    x1 = _leaky(raw_ref[0], NEG2)
    m = jnp.mean(x1, axis=0, keepdims=True)
    v = jnp.mean((x1 - m) ** 2, axis=0, keepdims=True)
    xh = (x1 - m) / jnp.sqrt(v + EPS)
    s = _dbf(style_ref[0], at_ref[...]) + ab_ref[...]
    cdim = x1.shape[1]
    out_ref[0] = s[:, :cdim] * xh + s[:, cdim:]


def _sc_gather(table, idx):
    from jax.experimental.pallas import tpu_sc as plsc
    bt = idx.shape[0]
    d = table.shape[1]
    nc, ns = 2, 16
    nw = nc * ns
    b_per_w = bt // nw
    ch = min(512, b_per_w)
    nch = b_per_w // ch
    mesh = plsc.VectorSubcoreMesh(core_axis_name="c", subcore_axis_name="s")

    @functools.partial(
        pl.kernel, mesh=mesh,
        out_type=jax.ShapeDtypeStruct((bt, d), jnp.float32),
        scratch_types=[
            pltpu.VMEM((ch,), jnp.int32),
            pltpu.VMEM((ch, d), jnp.float32),
            pltpu.SemaphoreType.DMA,
        ],
    )
    def gk(table_hbm, idx_hbm, out_hbm, idx_v, rows_v, sem):
        wid = jax.lax.axis_index("s") * nc + jax.lax.axis_index("c")
        base = wid * b_per_w
        for ci in range(nch):
            o = base + ci * ch
            pltpu.sync_copy(idx_hbm.at[pl.ds(o, ch)], idx_v)
            pltpu.async_copy(table_hbm.at[idx_v], rows_v, sem).wait()
            pltpu.sync_copy(rows_v, out_hbm.at[pl.ds(o, ch)])

    return gk(table, idx)


def _tail_kernel(x2_ref, g1t_ref, g1b_ref, g2t_ref, g2b_ref, ata_ref, atb_ref,
                 apa_ref, apb_ref, aga_ref, agb_ref, aota_ref, aotb_ref,
                 m1a_ref, m1fg_ref, m1b_ref, gam_ref, m2t_ref, m2b_ref,
                 m3t_ref, m3b_ref, wh_ref, bh_ref, out_ref):
    x2 = x2_ref[0]
    n = x2.shape[0]
    gam = gam_ref[...]
    fgm = jnp.max(x2, axis=0, keepdims=True)
    fg1 = _leaky((_dbf(fgm, g1t_ref[...]) + g1b_ref[...]) * _BN, NEG)
    fg2 = _leaky((_dbf(fg1, g2t_ref[...]) + g2b_ref[...]) * _BN, NEG)
    theta = _dbf(x2, ata_ref[...]) + _dbf(fg2, atb_ref[...])
    phi = _dbf(x2, apa_ref[...]) + _dbf(fg2, apb_ref[...])
    gv = _dbf(x2, aga_ref[...]) + _dbf(fg2, agb_ref[...])
    logits = _dbft(theta, phi)
    e = jnp.exp(logits - jnp.max(logits, axis=1, keepdims=True))
    beta = e / jnp.sum(e, axis=1, keepdims=True)
    o_pre = _dbf(beta, gv)
    feat_a = gam * _dbf(o_pre, aota_ref[...]) + x2
    feat_b = (gam * _dbf(o_pre, aotb_ref[...])
              + jnp.broadcast_to(fg2, (n, fg2.shape[1])))
    m1in = _dbf(feat_a, m1a_ref[...]) + _dbf(feat_b, m1fg_ref[...]) + m1b_ref[...]
    h = jax.nn.silu(m1in)
    h = jax.nn.silu(_dbf(h, m2t_ref[...]) + m2b_ref[...])
    h = jax.nn.silu(_dbf(h, m3t_ref[...]) + m3b_ref[...])
    raw = _dbf(h, wh_ref[...]) + bh_ref[...]
    col = jax.lax.broadcasted_iota(jnp.int32, raw.shape, 1)
    sc = jnp.clip(jnp.exp(raw), 0.0, 0.02)
    nrm = jnp.sqrt(jnp.sum(raw * raw, axis=0, keepdims=True))
    rot = raw / jnp.maximum(nrm, 1e-12)
    sg = jax.nn.sigmoid(raw)
    xyz = jnp.tanh(raw * 0.1) * 0.6
    out_ref[0] = jnp.where(col < 3, sc,
                 jnp.where(col < 7, rot,
                 jnp.where(col < 11, sg, xyz)))


def _bb(shape):
    nz = len(shape)
    return pl.BlockSpec((1,) + shape, lambda b, _n=nz: (b,) + (0,) * _n)


def _bf(arr):
    nd = arr.ndim
    return pl.BlockSpec(arr.shape, lambda b, _n=nd: (0,) * _n)


_CP = pltpu.CompilerParams(vmem_limit_bytes=63 * 1024 * 1024)


def kernel(x, z, params):
    p = params
    B, N, _ = x.shape
    f32 = jnp.float32
    k = 16
    blk = 512 if N % 512 == 0 else N

    def t(w):
        return jnp.asarray(w, f32).T

    def rb(b):
        return jnp.asarray(b, f32).reshape(1, -1)

    h1a = t(p['h1_W'][:, :3]); h1b = t(p['h1_W'][:, 3:]); h1bb = rb(p['h1_b'])
    h2t = t(p['h2_W']); h2b = rb(p['h2_b'])
    p1t = t(p['p1_W']); p1b = rb(p['p1_b'])
    p2t = t(p['p2_W']); p2b = rb(p['p2_b'])

    def edge_w(pref, apref):
        return (t(p[pref + 'w1_W']), rb(p[pref + 'w1_b']),
                t(p[pref + 'w2_W']), rb(p[pref + 'w2_b']),
                t(p[pref + 'x_W'][:, :128]), t(p[pref + 'x_W'][:, 128:]),
                rb(p[pref + 'x_b']),
                jnp.transpose(p[pref + 'o_W'], (2, 1, 0)).reshape(-1, 128),
                rb(p[pref + 'o_b']),
                t(p[apref + '_W']), rb(p[apref + '_b']))

    g1t = t(p['g1_W']); g1b = rb(p['g1_b'])
    g2t = t(p['g2_W']); g2b = rb(p['g2_b'])
    ata = t(p['at_W'][:, :128]); atb = t(p['at_W'][:, 128:])
    apa = t(p['ap_W'][:, :128]); apb = t(p['ap_W'][:, 128:])
    aga = t(p['ag_W'][:, :128]); agb = t(p['ag_W'][:, 128:])
    aota = t(p['ao_W'][:128, :]); aotb = t(p['ao_W'][128:, :])
    m1a = t(p['m1_W'][:, :128]); m1fg = t(p['m1_W'][:, 128:]); m1b = rb(p['m1_b'])
    gam = jnp.asarray(p['a_gamma'], f32).reshape(1, 1)
    m2t = t(p['m2_W']); m2b = rb(p['m2_b'])
    m3t = t(p['m3_W']); m3b = rb(p['m3_b'])
    wh = t(jnp.concatenate([p['ds_W'], p['dr_W'], p['do_W'], p['dh_W'],
                            p['dx_W']], axis=0))
    bh = rb(jnp.concatenate([p['ds_b'], p['dr_b'], p['do_b'], p['dh_b'],
                             p['dx_b']], axis=0))

    head_in = (x, z, h1a, h1b, h1bb, h2t, h2b, p1t, p1b, p2t, p2b)
    style, pc = pl.pallas_call(
        _head_kernel,
        grid=(B,),
        in_specs=[_bb((N, 3)), _bb((N, 128))] + [_bf(a) for a in head_in[2:]],
        out_specs=[_bb((N, 128)), _bb((N, 128))],
        out_shape=[jax.ShapeDtypeStruct((B, N, 128), f32)] * 2,
        compiler_params=_CP,
    )(*head_in)

    def extract_half(xin):
        bh_ = xin.shape[0]
        return pl.pallas_call(
            functools.partial(_extract_kernel, blk=blk, k=k),
            grid=(bh_,),
            in_specs=[_bb((N, 128))],
            out_specs=_bb((N, k)),
            out_shape=jax.ShapeDtypeStruct((bh_, N, k), jnp.int32),
            scratch_shapes=[pltpu.VMEM((blk, N), f32)],
            compiler_params=_CP,
        )(xin)

    def conv_half(xin, nb4, wts):
        bh_ = xin.shape[0]
        nblk = N // blk
        return pl.pallas_call(
            functools.partial(_conv_kernel, blk=blk, k=k),
            grid=(bh_, nblk),
            in_specs=[pl.BlockSpec((1, k, blk, 128),
                                   lambda b, i: (b, 0, i, 0)),
                      pl.BlockSpec((1, blk, 128), lambda b, i: (b, i, 0))]
                     + [pl.BlockSpec(a.shape,
                                     lambda b, i, _n=a.ndim: (0,) * _n)
                        for a in wts[:9]],
            out_specs=pl.BlockSpec((1, blk, 128), lambda b, i: (b, i, 0)),
            out_shape=jax.ShapeDtypeStruct((bh_, N, 128), f32),
            compiler_params=_CP,
        )(nb4, xin, *wts[:9])

    def edge(xin, wts):
        bfull = xin.shape[0]
        halves = [xin[:bfull // 2], xin[bfull // 2:]]
        idxs = [extract_half(h) for h in halves]
        raws = []
        for h, idx in zip(halves, idxs):
            bh_ = h.shape[0]
            idx_flat = jnp.transpose(idx, (0, 2, 1)).reshape(-1)
            nb = _sc_gather(h.reshape(bh_ * N, 128), idx_flat)
            raws.append(conv_half(h, nb.reshape(bh_, k, N, 128), wts))
        raw = jnp.concatenate(raws, axis=0)
        return pl.pallas_call(
            _adain_kernel,
            grid=(bfull,),
            in_specs=[_bb((N, 128)), _bb((N, 128)), _bf(wts[9]), _bf(wts[10])],
            out_specs=_bb((N, 128)),
            out_shape=jax.ShapeDtypeStruct((bfull, N, 128), f32),
            compiler_params=_CP,
        )(raw, style, wts[9], wts[10])

    x1 = edge(pc, edge_w('e1_', 'a1'))
    x2 = edge(x1, edge_w('e2_', 'a2'))

    tail_w = (g1t, g1b, g2t, g2b, ata, atb, apa, apb, aga, agb, aota, aotb,
              m1a, m1fg, m1b, gam, m2t, m2b, m3t, m3b, wh, bh)
    out = pl.pallas_call(
        _tail_kernel,
        grid=(B,),
        in_specs=[_bb((N, 128))] + [_bf(a) for a in tail_w],
        out_specs=_bb((N, 14)),
        out_shape=jax.ShapeDtypeStruct((B, N, 14), f32),
        compiler_params=_CP,
    )(x2, *tail_w)
    return out
